# baseline (speedup 1.0000x reference)
import functools

import jax
import jax.numpy as jnp
from jax import lax
from jax.experimental import pallas as pl
from jax.experimental.pallas import tpu as pltpu
from jax.experimental.pallas import tpu_sc as plsc

MAXB = 256
NFEAT = 64
NC, NS, LANES = 2, 16, 16
NW = NC * NS
HIST = NFEAT * MAXB


def _sc_partial_hists(Xt, gradient, hessian):
    N = Xt.shape[1]
    per_w = N // NW
    CH = 256
    n_ch = per_w // CH
    mesh = plsc.VectorSubcoreMesh(
        core_axis_name="c", subcore_axis_name="s", num_cores=NC, num_subcores=NS
    )

    @functools.partial(
        pl.kernel,
        out_type=(
            jax.ShapeDtypeStruct((NW, HIST), jnp.float32),
            jax.ShapeDtypeStruct((NW, HIST), jnp.float32),
        ),
        mesh=mesh,
        compiler_params=pltpu.CompilerParams(needs_layout_passes=False),
        scratch_types=[
            pltpu.VMEM((2, NFEAT, CH), jnp.int32),
            pltpu.VMEM((per_w,), jnp.float32),
            pltpu.VMEM((per_w,), jnp.float32),
            pltpu.VMEM((HIST,), jnp.float32),
            pltpu.VMEM((HIST,), jnp.float32),
            pltpu.SemaphoreType.DMA,
            pltpu.SemaphoreType.DMA,
            pltpu.SemaphoreType.DMA,
        ],
    )
    def hist_kernel(x_hbm, g_hbm, h_hbm, og_hbm, oh_hbm, xv, gv, hv, hg, hh,
                    sem0, sem1, semw):
        wid = lax.axis_index("s") * NC + lax.axis_index("c")
        base = wid * per_w
        sems = (sem0, sem1)

        wg = pltpu.async_copy(g_hbm.at[pl.ds(base, per_w)], gv, semw)
        wh = pltpu.async_copy(h_hbm.at[pl.ds(base, per_w)], hv, semw)
        pltpu.async_copy(x_hbm.at[:, pl.ds(base, CH)], xv.at[0], sem0)
        pltpu.async_copy(x_hbm.at[:, pl.ds(base + CH, CH)], xv.at[1], sem1)

        zeros = jnp.zeros((LANES,), jnp.float32)

        @pl.loop(0, HIST // LANES)
        def _zero(j):
            hg[pl.ds(j * LANES, LANES)] = zeros
            hh[pl.ds(j * LANES, LANES)] = zeros

        wg.wait()
        wh.wait()

        @pl.loop(0, n_ch, step=2)
        def _chunk(c):
            for b in range(2):
                cc = c + b
                pltpu.make_async_copy(
                    x_hbm.at[:, pl.ds(0, CH)], xv.at[b], sems[b]
                ).wait()

                @plsc.parallel_loop(0, CH // LANES, unroll=2)
                def _blk(blk):
                    i0 = blk * LANES
                    gblk = gv[pl.ds(cc * CH + i0, LANES)]
                    hblk = hv[pl.ds(cc * CH + i0, LANES)]
                    for f in range(NFEAT):
                        xvec = xv[b, f, pl.ds(i0, LANES)]
                        plsc.addupdate_scatter(
                            hg.at[pl.ds(f * MAXB, MAXB)], [xvec], gblk)
                        plsc.addupdate_scatter(
                            hh.at[pl.ds(f * MAXB, MAXB)], [xvec], hblk)

                @pl.when(cc + 2 < n_ch)
                def _prefetch():
                    s = base + (cc + 2) * CH
                    pltpu.async_copy(x_hbm.at[:, pl.ds(s, CH)], xv.at[b], sems[b])

        pltpu.sync_copy(hg, og_hbm.at[wid])
        pltpu.sync_copy(hh, oh_hbm.at[wid])

    return hist_kernel(Xt, gradient, hessian)


def _tc_merge_suffix(pg, ph):
    def body(pg_ref, ph_ref, og_ref, oh_ref):
        sg = jnp.sum(pg_ref[...], axis=0)
        sh = jnp.sum(ph_ref[...], axis=0)
        row = lax.broadcasted_iota(jnp.int32, (MAXB, MAXB), 0)
        col = lax.broadcasted_iota(jnp.int32, (MAXB, MAXB), 1)
        tri = (row > col).astype(jnp.float32)
        og_ref[...] = jnp.dot(sg, tri, preferred_element_type=jnp.float32)
        oh_ref[...] = jnp.dot(sh, tri, preferred_element_type=jnp.float32)

    return pl.pallas_call(
        body,
        out_shape=(
            jax.ShapeDtypeStruct((NFEAT, MAXB), jnp.float32),
            jax.ShapeDtypeStruct((NFEAT, MAXB), jnp.float32),
        ),
    )(pg, ph)


def kernel(X, gradient, hessian):
    Xt = X.T
    pg, ph = _sc_partial_hists(Xt, gradient, hessian)
    pg = pg.reshape(NW, NFEAT, MAXB)
    ph = ph.reshape(NW, NFEAT, MAXB)
    Gl, Hl = _tc_merge_suffix(pg, ph)
    return (Gl[None], Hl[None])


# speedup vs baseline: 1.0079x; 1.0079x over previous
import functools

import jax
import jax.numpy as jnp
from jax import lax
from jax.experimental import pallas as pl
from jax.experimental.pallas import tpu as pltpu
from jax.experimental.pallas import tpu_sc as plsc

MAXB = 256
NFEAT = 64
NC, NS, LANES = 2, 16, 16
NW = NC * NS
HIST = NFEAT * MAXB


def _sc_partial_hists(Xt, gradient, hessian):
    N = Xt.shape[1]
    per_w = N // NW
    CH = 256
    n_ch = per_w // CH
    mesh = plsc.VectorSubcoreMesh(
        core_axis_name="c", subcore_axis_name="s", num_cores=NC, num_subcores=NS
    )

    @functools.partial(
        pl.kernel,
        out_type=(
            jax.ShapeDtypeStruct((NW, HIST), jnp.float32),
            jax.ShapeDtypeStruct((NW, HIST), jnp.float32),
        ),
        mesh=mesh,
        compiler_params=pltpu.CompilerParams(needs_layout_passes=False),
        scratch_types=[
            pltpu.VMEM((2, NFEAT, CH), jnp.int32),
            pltpu.VMEM((per_w,), jnp.float32),
            pltpu.VMEM((per_w,), jnp.float32),
            pltpu.VMEM((HIST,), jnp.float32),
            pltpu.VMEM((HIST,), jnp.float32),
            pltpu.SemaphoreType.DMA,
            pltpu.SemaphoreType.DMA,
            pltpu.SemaphoreType.DMA,
        ],
    )
    def hist_kernel(x_hbm, g_hbm, h_hbm, og_hbm, oh_hbm, xv, gv, hv, hg, hh,
                    sem0, sem1, semw):
        wid = lax.axis_index("s") * NC + lax.axis_index("c")
        base = wid * per_w
        sems = (sem0, sem1)

        wg = pltpu.async_copy(g_hbm.at[pl.ds(base, per_w)], gv, semw)
        wh = pltpu.async_copy(h_hbm.at[pl.ds(base, per_w)], hv, semw)
        pltpu.async_copy(x_hbm.at[:, pl.ds(base, CH)], xv.at[0], sem0)
        pltpu.async_copy(x_hbm.at[:, pl.ds(base + CH, CH)], xv.at[1], sem1)

        zeros = jnp.zeros((LANES,), jnp.float32)

        @pl.loop(0, HIST // LANES)
        def _zero(j):
            hg[pl.ds(j * LANES, LANES)] = zeros
            hh[pl.ds(j * LANES, LANES)] = zeros

        wg.wait()
        wh.wait()

        @pl.loop(0, n_ch, step=2)
        def _chunk(c):
            for b in range(2):
                cc = c + b
                pltpu.make_async_copy(
                    x_hbm.at[:, pl.ds(0, CH)], xv.at[b], sems[b]
                ).wait()

                @plsc.parallel_loop(0, CH // LANES, unroll=4)
                def _blk(blk):
                    i0 = blk * LANES
                    gblk = gv[pl.ds(cc * CH + i0, LANES)]
                    hblk = hv[pl.ds(cc * CH + i0, LANES)]
                    for f in range(NFEAT):
                        xvec = xv[b, f, pl.ds(i0, LANES)]
                        plsc.addupdate_scatter(
                            hg.at[pl.ds(f * MAXB, MAXB)], [xvec], gblk)
                        plsc.addupdate_scatter(
                            hh.at[pl.ds(f * MAXB, MAXB)], [xvec], hblk)

                @pl.when(cc + 2 < n_ch)
                def _prefetch():
                    s = base + (cc + 2) * CH
                    pltpu.async_copy(x_hbm.at[:, pl.ds(s, CH)], xv.at[b], sems[b])

        pltpu.sync_copy(hg, og_hbm.at[wid])
        pltpu.sync_copy(hh, oh_hbm.at[wid])

    return hist_kernel(Xt, gradient, hessian)


def _tc_merge_suffix(pg, ph):
    def body(pg_ref, ph_ref, og_ref, oh_ref):
        sg = jnp.sum(pg_ref[...], axis=0)
        sh = jnp.sum(ph_ref[...], axis=0)
        row = lax.broadcasted_iota(jnp.int32, (MAXB, MAXB), 0)
        col = lax.broadcasted_iota(jnp.int32, (MAXB, MAXB), 1)
        tri = (row > col).astype(jnp.float32)
        og_ref[...] = jnp.dot(sg, tri, preferred_element_type=jnp.float32)
        oh_ref[...] = jnp.dot(sh, tri, preferred_element_type=jnp.float32)

    return pl.pallas_call(
        body,
        out_shape=(
            jax.ShapeDtypeStruct((NFEAT, MAXB), jnp.float32),
            jax.ShapeDtypeStruct((NFEAT, MAXB), jnp.float32),
        ),
    )(pg, ph)


def kernel(X, gradient, hessian):
    Xt = X.T
    pg, ph = _sc_partial_hists(Xt, gradient, hessian)
    pg = pg.reshape(NW, NFEAT, MAXB)
    ph = ph.reshape(NW, NFEAT, MAXB)
    Gl, Hl = _tc_merge_suffix(pg, ph)
    return (Gl[None], Hl[None])
